# Initial kernel scaffold; baseline (speedup 1.0000x reference)
import jax
import jax.numpy as jnp
from jax.experimental import pallas as pl


def kernel(x, edge_index, edge_type, rel_weight, self_w, self_b):
    raise NotImplementedError("write your pallas kernel here")


# speedup vs baseline: 15.6655x; 15.6655x over previous
import jax
import jax.numpy as jnp
from jax import lax
from jax.experimental import pallas as pl
from jax.experimental.pallas import tpu as pltpu
from jax.experimental.pallas import tpu_sc as plsc

N = 10000
E = 320000
D = 128
R = 8

NC = 2
NS = 16
DH = D // NC
EPW = E // NS
C = 80
NCHUNK = EPW // C
NP = 10240
RPT = NP // NS
BN = 1000


def _y_body(x_ref, w_ref, y0_ref, y1_ref):
    for r in range(R):
        h = lax.dot_general(
            x_ref[...], w_ref[r], (((1,), (0,)), ((), ())),
            preferred_element_type=jnp.float32)
        y0_ref[r] = h[:, :DH]
        y1_ref[r] = h[:, DH:]


def _compute_y(x, w):
    return pl.pallas_call(
        _y_body,
        grid=(N // BN,),
        in_specs=[
            pl.BlockSpec((BN, D), lambda i: (i, 0)),
            pl.BlockSpec((R, D, D), lambda i: (0, 0, 0)),
        ],
        out_specs=[
            pl.BlockSpec((R, BN, DH), lambda i: (0, i, 0)),
            pl.BlockSpec((R, BN, DH), lambda i: (0, i, 0)),
        ],
        out_shape=[
            jax.ShapeDtypeStruct((R, N, DH), jnp.float32),
            jax.ShapeDtypeStruct((R, N, DH), jnp.float32),
        ],
    )(x, w)


def _sc_body(y0_hbm, y1_hbm, src_hbm, et_hbm, dst_hbm, z2_hbm, z1_hbm,
             agg_out, deg_out,
             src_v, et_v, idx_v, dst_v, rows_v, ones_v, agg_sh, deg_sh, sem):
    cid = lax.axis_index("c")
    sid = lax.axis_index("s")

    slab = pl.ds(sid * RPT, RPT)
    pltpu.sync_copy(z2_hbm.at[slab], agg_sh.at[slab])

    @pl.when(cid == 0)
    def _():
        pltpu.sync_copy(z1_hbm.at[slab], deg_sh.at[slab])

    pltpu.sync_copy(src_hbm.at[sid], src_v)
    pltpu.sync_copy(et_hbm.at[sid], et_v)
    pltpu.sync_copy(dst_hbm.at[sid], dst_v)

    for i in range(C // 16):
        ones_v[pl.ds(i * 16, 16)] = jnp.ones((16,), jnp.float32)

    def idx_body(i, carry):
        s = pl.ds(i * 16, 16)
        idx_v[s] = et_v[s] * N + src_v[s]
        return carry
    lax.fori_loop(0, EPW // 16, idx_body, 0)

    plsc.subcore_barrier()

    def chunk_body0(j, carry):
        pltpu.async_copy(y0_hbm.at[idx_v.at[pl.ds(j * C, C)]], rows_v, sem).wait()
        pltpu.sync_copy(rows_v, agg_sh.at[dst_v.at[j]], add=True)
        pltpu.sync_copy(ones_v, deg_sh.at[dst_v.at[j]], add=True)
        return carry

    def chunk_body1(j, carry):
        pltpu.async_copy(y1_hbm.at[idx_v.at[pl.ds(j * C, C)]], rows_v, sem).wait()
        pltpu.sync_copy(rows_v, agg_sh.at[dst_v.at[j]], add=True)
        return carry

    @pl.when(cid == 0)
    def _():
        lax.fori_loop(0, NCHUNK, chunk_body0, 0)

    @pl.when(cid == 1)
    def _():
        lax.fori_loop(0, NCHUNK, chunk_body1, 0)

    plsc.subcore_barrier()

    pltpu.sync_copy(agg_sh.at[slab], agg_out.at[cid, slab])

    @pl.when(cid == 0)
    def _():
        pltpu.sync_copy(deg_sh.at[slab], deg_out.at[slab])


def _sc_aggregate(y0, y1, src2, et2, dst3, z2, z1):
    mesh = plsc.VectorSubcoreMesh(core_axis_name="c", subcore_axis_name="s",
                                  num_cores=NC, num_subcores=NS)
    k = pl.kernel(
        _sc_body,
        out_type=(jax.ShapeDtypeStruct((NC, NP, DH), jnp.float32),
                  jax.ShapeDtypeStruct((NP,), jnp.float32)),
        mesh=mesh,
        scratch_types=[
            pltpu.VMEM((EPW,), jnp.int32),
            pltpu.VMEM((EPW,), jnp.int32),
            pltpu.VMEM((EPW,), jnp.int32),
            pltpu.VMEM((NCHUNK, C), jnp.int32),
            pltpu.VMEM((C, DH), jnp.float32),
            pltpu.VMEM((C,), jnp.float32),
            pltpu.VMEM_SHARED((NP, DH), jnp.float32),
            pltpu.VMEM_SHARED((NP,), jnp.float32),
            pltpu.SemaphoreType.DMA,
        ],
        compiler_params=pltpu.CompilerParams(use_tc_tiling_on_sc=False),
    )
    return k(y0, y1, src2, et2, dst3, z2, z1)


def _combine_body(x_ref, w_ref, b_ref, pa_ref, pd_ref, o_ref):
    agg = jnp.concatenate([pa_ref[0], pa_ref[1]], axis=-1)
    deg = jnp.maximum(pd_ref[...], 1.0)
    h = lax.dot_general(x_ref[...], w_ref[...], (((1,), (1,)), ((), ())),
                        preferred_element_type=jnp.float32)
    o_ref[...] = jnp.maximum(h + b_ref[...] + agg / deg, 0.0)


def _combine(x, self_w, self_b, pa, pd):
    return pl.pallas_call(
        _combine_body,
        grid=(N // BN,),
        in_specs=[
            pl.BlockSpec((BN, D), lambda i: (i, 0)),
            pl.BlockSpec((D, D), lambda i: (0, 0)),
            pl.BlockSpec((1, D), lambda i: (0, 0)),
            pl.BlockSpec((NC, BN, DH), lambda i: (0, i, 0)),
            pl.BlockSpec((BN, 1), lambda i: (i, 0)),
        ],
        out_specs=pl.BlockSpec((BN, D), lambda i: (i, 0)),
        out_shape=jax.ShapeDtypeStruct((N, D), jnp.float32),
    )(x, self_w, self_b.reshape(1, D), pa, pd)


def kernel(x, edge_index, edge_type, rel_weight, self_w, self_b):
    src2 = edge_index[0].astype(jnp.int32).reshape(NS, EPW)
    dst3 = edge_index[1].astype(jnp.int32).reshape(NS, NCHUNK, C)
    et2 = edge_type.astype(jnp.int32).reshape(NS, EPW)

    y0, y1 = _compute_y(x, rel_weight)
    y0f = y0.reshape(R * N, DH)
    y1f = y1.reshape(R * N, DH)

    z2 = jnp.zeros((NP, DH), jnp.float32)
    z1 = jnp.zeros((NP,), jnp.float32)
    pa, pd = _sc_aggregate(y0f, y1f, src2, et2, dst3, z2, z1)

    return _combine(x, self_w, self_b, pa, pd.reshape(NP, 1))


# speedup vs baseline: 22.4986x; 1.4362x over previous
import jax
import jax.numpy as jnp
from jax import lax
from jax.experimental import pallas as pl
from jax.experimental.pallas import tpu as pltpu
from jax.experimental.pallas import tpu_sc as plsc

N = 10000
E = 320000
D = 128
R = 8

NC = 2
NS = 16
DH = D // NC
C = 128
NCHUNK = 157
EPW = NCHUNK * C
EPAD = NS * EPW
NP = 10240
RPT = NP // NS
BN = 1000


def _y_body(x_ref, w_ref, y0_ref, y1_ref):
    for r in range(R):
        h = lax.dot_general(
            x_ref[...], w_ref[r], (((1,), (0,)), ((), ())),
            preferred_element_type=jnp.float32)
        y0_ref[r] = h[:, :DH]
        y1_ref[r] = h[:, DH:]


def _compute_y(x, w):
    return pl.pallas_call(
        _y_body,
        grid=(N // BN,),
        in_specs=[
            pl.BlockSpec((BN, D), lambda i: (i, 0)),
            pl.BlockSpec((R, D, D), lambda i: (0, 0, 0)),
        ],
        out_specs=[
            pl.BlockSpec((R, BN, DH), lambda i: (0, i, 0)),
            pl.BlockSpec((R, BN, DH), lambda i: (0, i, 0)),
        ],
        out_shape=[
            jax.ShapeDtypeStruct((R, N, DH), jnp.float32),
            jax.ShapeDtypeStruct((R, N, DH), jnp.float32),
        ],
    )(x, w)


def _edge_prep_body(src_ref, et_ref, idx_ref):
    idx_ref[...] = et_ref[...] * N + src_ref[...]


def _edge_prep(src2, et2):
    return pl.pallas_call(
        _edge_prep_body,
        out_shape=jax.ShapeDtypeStruct((NS, EPW), jnp.int32),
    )(src2, et2)


def _sc_body(y0_hbm, y1_hbm, idx_hbm, dst_hbm, z2_hbm, z1_hbm,
             agg_out, deg_out,
             idx_v, dst_v, rows_v, ones_v, agg_sh, deg_sh, sem):
    cid = lax.axis_index("c")
    sid = lax.axis_index("s")

    slab = pl.ds(sid * RPT, RPT)
    pltpu.sync_copy(z2_hbm.at[slab], agg_sh.at[slab])

    @pl.when(cid == 1)
    def _():
        pltpu.sync_copy(z1_hbm.at[slab], deg_sh.at[slab])

    pltpu.sync_copy(idx_hbm.at[sid], idx_v)
    pltpu.sync_copy(dst_hbm.at[sid], dst_v)

    for i in range(C // 16):
        ones_v[pl.ds(i * 16, 16)] = jnp.ones((16,), jnp.float32)

    plsc.subcore_barrier()

    def run(y_hbm, with_deg):
        def start_gather(j, p):
            pltpu.async_copy(
                y_hbm.at[idx_v.at[pl.ds(j * C, C)]], rows_v.at[p], sem)

        def wait_gather(j, p):
            pltpu.make_async_copy(
                y_hbm.at[idx_v.at[pl.ds(j * C, C)]], rows_v.at[p], sem).wait()

        def scatter(j, p):
            pltpu.sync_copy(rows_v.at[p], agg_sh.at[dst_v.at[j]], add=True)
            if with_deg:
                pltpu.sync_copy(ones_v, deg_sh.at[dst_v.at[j]], add=True)

        start_gather(0, 0)

        def body(j, carry):
            p = lax.bitwise_and(j, 1)
            start_gather(j + 1, 1 - p)
            wait_gather(j, p)
            scatter(j, p)
            return carry
        lax.fori_loop(0, NCHUNK - 1, body, 0)

        last = NCHUNK - 1
        p = last % 2
        wait_gather(last, p)
        scatter(last, p)

    @pl.when(cid == 0)
    def _():
        run(y0_hbm, False)

    @pl.when(cid == 1)
    def _():
        run(y1_hbm, True)

    plsc.subcore_barrier()

    pltpu.sync_copy(agg_sh.at[slab], agg_out.at[cid, slab])

    @pl.when(cid == 1)
    def _():
        pltpu.sync_copy(deg_sh.at[slab], deg_out.at[slab])


def _sc_aggregate(y0, y1, idx2, dst3, z2, z1):
    mesh = plsc.VectorSubcoreMesh(core_axis_name="c", subcore_axis_name="s",
                                  num_cores=NC, num_subcores=NS)
    k = pl.kernel(
        _sc_body,
        out_type=(jax.ShapeDtypeStruct((NC, NP, DH), jnp.float32),
                  jax.ShapeDtypeStruct((NP,), jnp.float32)),
        mesh=mesh,
        scratch_types=[
            pltpu.VMEM((EPW,), jnp.int32),
            pltpu.VMEM((NCHUNK, C), jnp.int32),
            pltpu.VMEM((2, C, DH), jnp.float32),
            pltpu.VMEM((C,), jnp.float32),
            pltpu.VMEM_SHARED((NP, DH), jnp.float32),
            pltpu.VMEM_SHARED((NP,), jnp.float32),
            pltpu.SemaphoreType.DMA,
        ],
        compiler_params=pltpu.CompilerParams(use_tc_tiling_on_sc=False),
    )
    return k(y0, y1, idx2, dst3, z2, z1)


def _combine_body(x_ref, w_ref, b_ref, pa_ref, pd_ref, o_ref):
    agg = jnp.concatenate([pa_ref[0], pa_ref[1]], axis=-1)
    deg = jnp.maximum(pd_ref[...], 1.0)
    h = lax.dot_general(x_ref[...], w_ref[...], (((1,), (1,)), ((), ())),
                        preferred_element_type=jnp.float32)
    o_ref[...] = jnp.maximum(h + b_ref[...] + agg / deg, 0.0)


def _combine(x, self_w, self_b, pa, pd):
    return pl.pallas_call(
        _combine_body,
        grid=(N // BN,),
        in_specs=[
            pl.BlockSpec((BN, D), lambda i: (i, 0)),
            pl.BlockSpec((D, D), lambda i: (0, 0)),
            pl.BlockSpec((1, D), lambda i: (0, 0)),
            pl.BlockSpec((NC, BN, DH), lambda i: (0, i, 0)),
            pl.BlockSpec((BN, 1), lambda i: (i, 0)),
        ],
        out_specs=pl.BlockSpec((BN, D), lambda i: (i, 0)),
        out_shape=jax.ShapeDtypeStruct((N, D), jnp.float32),
    )(x, self_w, self_b.reshape(1, D), pa, pd)


def kernel(x, edge_index, edge_type, rel_weight, self_w, self_b):
    src = edge_index[0].astype(jnp.int32)
    dst = edge_index[1].astype(jnp.int32)
    et = edge_type.astype(jnp.int32)

    pad = EPAD - E
    zpad = jnp.zeros((pad,), jnp.int32)
    src2 = jnp.concatenate([src, zpad]).reshape(NS, EPW)
    et2 = jnp.concatenate([et, zpad]).reshape(NS, EPW)
    dst3 = jnp.concatenate([dst, jnp.full((pad,), NP - 1, jnp.int32)]
                           ).reshape(NS, NCHUNK, C)

    idx2 = _edge_prep(src2, et2)
    y0, y1 = _compute_y(x, rel_weight)
    y0f = y0.reshape(R * N, DH)
    y1f = y1.reshape(R * N, DH)

    z2 = jnp.zeros((NP, DH), jnp.float32)
    z1 = jnp.zeros((NP,), jnp.float32)
    pa, pd = _sc_aggregate(y0f, y1f, idx2, dst3, z2, z1)

    return _combine(x, self_w, self_b, pa, pd.reshape(NP, 1))


# speedup vs baseline: 23.5976x; 1.0488x over previous
import jax
import jax.numpy as jnp
from jax import lax
from jax.experimental import pallas as pl
from jax.experimental.pallas import tpu as pltpu
from jax.experimental.pallas import tpu_sc as plsc

N = 10000
E = 320000
D = 128
R = 8

NC = 2
NS = 16
DH = D // NC
C = 128
NCHUNK = 157
EPW = NCHUNK * C
EPAD = NS * EPW
NP = 10240
RPT = NP // NS
BN = 1000


def _y_body(x_ref, w_ref, y0_ref, y1_ref):
    for r in range(R):
        h = lax.dot_general(
            x_ref[...], w_ref[r], (((1,), (0,)), ((), ())),
            preferred_element_type=jnp.float32)
        y0_ref[r] = h[:, :DH]
        y1_ref[r] = h[:, DH:]


def _compute_y(x, w):
    return pl.pallas_call(
        _y_body,
        grid=(N // BN,),
        in_specs=[
            pl.BlockSpec((BN, D), lambda i: (i, 0)),
            pl.BlockSpec((R, D, D), lambda i: (0, 0, 0)),
        ],
        out_specs=[
            pl.BlockSpec((R, BN, DH), lambda i: (0, i, 0)),
            pl.BlockSpec((R, BN, DH), lambda i: (0, i, 0)),
        ],
        out_shape=[
            jax.ShapeDtypeStruct((R, N, DH), jnp.float32),
            jax.ShapeDtypeStruct((R, N, DH), jnp.float32),
        ],
    )(x, w)


def _edge_prep_body(src_ref, et_ref, idx_ref):
    idx_ref[...] = et_ref[...] * N + src_ref[...]


def _edge_prep(src2, et2):
    return pl.pallas_call(
        _edge_prep_body,
        out_shape=jax.ShapeDtypeStruct((NS, EPW), jnp.int32),
    )(src2, et2)


def _sc_body(y0_hbm, y1_hbm, idx_hbm, dst_hbm, z2_hbm, z1_hbm,
             agg_out, deg_out,
             idx_v, dst_v, rows_v, ones_v, agg_sh, deg_sh, sem, sem_s):
    cid = lax.axis_index("c")
    sid = lax.axis_index("s")

    slab = pl.ds(sid * RPT, RPT)
    pltpu.sync_copy(z2_hbm.at[slab], agg_sh.at[slab])

    @pl.when(cid == 1)
    def _():
        pltpu.sync_copy(z1_hbm.at[slab], deg_sh.at[slab])

    pltpu.sync_copy(idx_hbm.at[sid], idx_v)
    pltpu.sync_copy(dst_hbm.at[sid], dst_v)

    for i in range(C // 16):
        ones_v[pl.ds(i * 16, 16)] = jnp.ones((16,), jnp.float32)

    plsc.subcore_barrier()

    def run(y_hbm, with_deg):
        def start_gather(j, p):
            pltpu.async_copy(
                y_hbm.at[idx_v.at[pl.ds(j * C, C)]], rows_v.at[p], sem)

        def wait_gather(j, p):
            pltpu.make_async_copy(
                y_hbm.at[idx_v.at[pl.ds(j * C, C)]], rows_v.at[p], sem).wait()

        def start_scatter(j, p):
            pltpu.async_copy(rows_v.at[p], agg_sh.at[dst_v.at[j]], sem_s,
                             add=True)
            if with_deg:
                pltpu.async_copy(ones_v, deg_sh.at[dst_v.at[j]], sem_s,
                                 add=True)

        def wait_scatter(j, p):
            pltpu.make_async_copy(rows_v.at[p], agg_sh.at[dst_v.at[j]],
                                  sem_s).wait()
            if with_deg:
                pltpu.make_async_copy(ones_v, deg_sh.at[dst_v.at[j]],
                                      sem_s).wait()

        start_gather(0, 0)
        start_gather(1, 1)

        def body(j, carry):
            p = lax.bitwise_and(j, 3)
            wait_gather(j, p)
            start_scatter(j, p)
            pl.when(j >= 2)(lambda: wait_scatter(j - 2,
                                                 lax.bitwise_and(j - 2, 3)))
            pl.when(j + 2 < NCHUNK)(
                lambda: start_gather(j + 2, lax.bitwise_and(j + 2, 3)))
            return carry
        lax.fori_loop(0, NCHUNK, body, 0)

        wait_scatter(NCHUNK - 2, (NCHUNK - 2) % 4)
        wait_scatter(NCHUNK - 1, (NCHUNK - 1) % 4)

    @pl.when(cid == 0)
    def _():
        run(y0_hbm, False)

    @pl.when(cid == 1)
    def _():
        run(y1_hbm, True)

    plsc.subcore_barrier()

    pltpu.sync_copy(agg_sh.at[slab], agg_out.at[cid, slab])

    @pl.when(cid == 1)
    def _():
        pltpu.sync_copy(deg_sh.at[slab], deg_out.at[slab])


def _sc_aggregate(y0, y1, idx2, dst3, z2, z1):
    mesh = plsc.VectorSubcoreMesh(core_axis_name="c", subcore_axis_name="s",
                                  num_cores=NC, num_subcores=NS)
    k = pl.kernel(
        _sc_body,
        out_type=(jax.ShapeDtypeStruct((NC, NP, DH), jnp.float32),
                  jax.ShapeDtypeStruct((NP,), jnp.float32)),
        mesh=mesh,
        scratch_types=[
            pltpu.VMEM((EPW,), jnp.int32),
            pltpu.VMEM((NCHUNK, C), jnp.int32),
            pltpu.VMEM((4, C, DH), jnp.float32),
            pltpu.VMEM((C,), jnp.float32),
            pltpu.VMEM_SHARED((NP, DH), jnp.float32),
            pltpu.VMEM_SHARED((NP,), jnp.float32),
            pltpu.SemaphoreType.DMA,
            pltpu.SemaphoreType.DMA,
        ],
        compiler_params=pltpu.CompilerParams(use_tc_tiling_on_sc=False),
    )
    return k(y0, y1, idx2, dst3, z2, z1)


def _combine_body(x_ref, w_ref, b_ref, pa_ref, pd_ref, o_ref):
    agg = jnp.concatenate([pa_ref[0], pa_ref[1]], axis=-1)
    deg = jnp.maximum(pd_ref[...], 1.0)
    h = lax.dot_general(x_ref[...], w_ref[...], (((1,), (1,)), ((), ())),
                        preferred_element_type=jnp.float32)
    o_ref[...] = jnp.maximum(h + b_ref[...] + agg / deg, 0.0)


def _combine(x, self_w, self_b, pa, pd):
    return pl.pallas_call(
        _combine_body,
        grid=(N // BN,),
        in_specs=[
            pl.BlockSpec((BN, D), lambda i: (i, 0)),
            pl.BlockSpec((D, D), lambda i: (0, 0)),
            pl.BlockSpec((1, D), lambda i: (0, 0)),
            pl.BlockSpec((NC, BN, DH), lambda i: (0, i, 0)),
            pl.BlockSpec((BN, 1), lambda i: (i, 0)),
        ],
        out_specs=pl.BlockSpec((BN, D), lambda i: (i, 0)),
        out_shape=jax.ShapeDtypeStruct((N, D), jnp.float32),
    )(x, self_w, self_b.reshape(1, D), pa, pd)


def kernel(x, edge_index, edge_type, rel_weight, self_w, self_b):
    src = edge_index[0].astype(jnp.int32)
    dst = edge_index[1].astype(jnp.int32)
    et = edge_type.astype(jnp.int32)

    pad = EPAD - E
    zpad = jnp.zeros((pad,), jnp.int32)
    src2 = jnp.concatenate([src, zpad]).reshape(NS, EPW)
    et2 = jnp.concatenate([et, zpad]).reshape(NS, EPW)
    dst3 = jnp.concatenate([dst, jnp.full((pad,), NP - 1, jnp.int32)]
                           ).reshape(NS, NCHUNK, C)

    idx2 = _edge_prep(src2, et2)
    y0, y1 = _compute_y(x, rel_weight)
    y0f = y0.reshape(R * N, DH)
    y1f = y1.reshape(R * N, DH)

    z2 = jnp.zeros((NP, DH), jnp.float32)
    z1 = jnp.zeros((NP,), jnp.float32)
    pa, pd = _sc_aggregate(y0f, y1f, idx2, dst3, z2, z1)

    return _combine(x, self_w, self_b, pa, pd.reshape(NP, 1))


# speedup vs baseline: 29.3632x; 1.2443x over previous
import jax
import jax.numpy as jnp
from jax import lax
from jax.experimental import pallas as pl
from jax.experimental.pallas import tpu as pltpu
from jax.experimental.pallas import tpu_sc as plsc

N = 10000
E = 320000
D = 128
R = 8

NC = 2
NS = 16
DH = D // NC
C = 128
NCHUNK = 157
EPW = NCHUNK * C
EPAD = NS * EPW
NP = 10240
RPT = NP // NS
BN = 1000


def _y_body(x_ref, w_ref, y_ref):
    for r in range(R):
        y_ref[r] = lax.dot_general(
            x_ref[...], w_ref[r], (((1,), (0,)), ((), ())),
            preferred_element_type=jnp.float32)


def _compute_y(x, w):
    return pl.pallas_call(
        _y_body,
        grid=(N // BN,),
        in_specs=[
            pl.BlockSpec((BN, D), lambda i: (i, 0)),
            pl.BlockSpec((R, D, D), lambda i: (0, 0, 0)),
        ],
        out_specs=pl.BlockSpec((R, BN, D), lambda i: (0, i, 0)),
        out_shape=jax.ShapeDtypeStruct((R, N, D), jnp.float32),
    )(x, w)


def _edge_prep_body(src_ref, et_ref, ia_ref, ib_ref):
    base = (et_ref[...] * N + src_ref[...]) * 2
    ia_ref[...] = base
    ib_ref[...] = base + 1


def _edge_prep(src2, et2):
    return pl.pallas_call(
        _edge_prep_body,
        out_shape=[
            jax.ShapeDtypeStruct((NS, EPW), jnp.int32),
            jax.ShapeDtypeStruct((NS, EPW), jnp.int32),
        ],
    )(src2, et2)


def _sc_body(y_hbm, ia_hbm, ib_hbm, dst_hbm, z2_hbm, z1_hbm,
             agg_out, deg_out,
             idx_v, dst_v, rows_v, ones_v, agg_sh, deg_sh, sem, sem_s):
    cid = lax.axis_index("c")
    sid = lax.axis_index("s")

    slab = pl.ds(sid * RPT, RPT)
    pltpu.sync_copy(z2_hbm.at[slab], agg_sh.at[slab])

    @pl.when(cid == 1)
    def _():
        pltpu.sync_copy(z1_hbm.at[slab], deg_sh.at[slab])

    @pl.when(cid == 0)
    def _():
        pltpu.sync_copy(ia_hbm.at[sid], idx_v)

    @pl.when(cid == 1)
    def _():
        pltpu.sync_copy(ib_hbm.at[sid], idx_v)

    pltpu.sync_copy(dst_hbm.at[sid], dst_v)

    for i in range(C // 16):
        ones_v[pl.ds(i * 16, 16)] = jnp.ones((16,), jnp.float32)

    plsc.subcore_barrier()

    def run(with_deg):
        def start_gather(j, p):
            pltpu.async_copy(
                y_hbm.at[idx_v.at[pl.ds(j * C, C)]], rows_v.at[p], sem)

        def wait_gather(j, p):
            pltpu.make_async_copy(
                y_hbm.at[idx_v.at[pl.ds(j * C, C)]], rows_v.at[p], sem).wait()

        def start_scatter(j, p):
            pltpu.async_copy(rows_v.at[p], agg_sh.at[dst_v.at[j]], sem_s,
                             add=True)
            if with_deg:
                pltpu.async_copy(ones_v, deg_sh.at[dst_v.at[j]], sem_s,
                                 add=True)

        def wait_scatter(j, p):
            pltpu.make_async_copy(rows_v.at[p], agg_sh.at[dst_v.at[j]],
                                  sem_s).wait()
            if with_deg:
                pltpu.make_async_copy(ones_v, deg_sh.at[dst_v.at[j]],
                                      sem_s).wait()

        start_gather(0, 0)
        start_gather(1, 1)

        def body(j, carry):
            p = lax.bitwise_and(j, 3)
            wait_gather(j, p)
            start_scatter(j, p)
            pl.when(j >= 2)(lambda: wait_scatter(j - 2,
                                                 lax.bitwise_and(j - 2, 3)))
            pl.when(j + 2 < NCHUNK)(
                lambda: start_gather(j + 2, lax.bitwise_and(j + 2, 3)))
            return carry
        lax.fori_loop(0, NCHUNK, body, 0)

        wait_scatter(NCHUNK - 2, (NCHUNK - 2) % 4)
        wait_scatter(NCHUNK - 1, (NCHUNK - 1) % 4)

    @pl.when(cid == 0)
    def _():
        run(False)

    @pl.when(cid == 1)
    def _():
        run(True)

    plsc.subcore_barrier()

    pltpu.sync_copy(agg_sh.at[slab], agg_out.at[cid, slab])

    @pl.when(cid == 1)
    def _():
        pltpu.sync_copy(deg_sh.at[slab], deg_out.at[slab])


def _sc_aggregate(yf, ia2, ib2, dst3, z2, z1):
    mesh = plsc.VectorSubcoreMesh(core_axis_name="c", subcore_axis_name="s",
                                  num_cores=NC, num_subcores=NS)
    k = pl.kernel(
        _sc_body,
        out_type=(jax.ShapeDtypeStruct((NC, NP, DH), jnp.float32),
                  jax.ShapeDtypeStruct((NP,), jnp.float32)),
        mesh=mesh,
        scratch_types=[
            pltpu.VMEM((EPW,), jnp.int32),
            pltpu.VMEM((NCHUNK, C), jnp.int32),
            pltpu.VMEM((4, C, DH), jnp.float32),
            pltpu.VMEM((C,), jnp.float32),
            pltpu.VMEM_SHARED((NP, DH), jnp.float32),
            pltpu.VMEM_SHARED((NP,), jnp.float32),
            pltpu.SemaphoreType.DMA,
            pltpu.SemaphoreType.DMA,
        ],
        compiler_params=pltpu.CompilerParams(use_tc_tiling_on_sc=False),
    )
    return k(yf, ia2, ib2, dst3, z2, z1)


def _combine_body(x_ref, w_ref, b_ref, pa_ref, pd_ref, o_ref):
    agg = jnp.concatenate([pa_ref[0], pa_ref[1]], axis=-1)
    deg = jnp.maximum(pd_ref[...], 1.0)
    h = lax.dot_general(x_ref[...], w_ref[...], (((1,), (1,)), ((), ())),
                        preferred_element_type=jnp.float32)
    o_ref[...] = jnp.maximum(h + b_ref[...] + agg / deg, 0.0)


def _combine(x, self_w, self_b, pa, pd):
    return pl.pallas_call(
        _combine_body,
        grid=(N // BN,),
        in_specs=[
            pl.BlockSpec((BN, D), lambda i: (i, 0)),
            pl.BlockSpec((D, D), lambda i: (0, 0)),
            pl.BlockSpec((1, D), lambda i: (0, 0)),
            pl.BlockSpec((NC, BN, DH), lambda i: (0, i, 0)),
            pl.BlockSpec((BN, 1), lambda i: (i, 0)),
        ],
        out_specs=pl.BlockSpec((BN, D), lambda i: (i, 0)),
        out_shape=jax.ShapeDtypeStruct((N, D), jnp.float32),
    )(x, self_w, self_b.reshape(1, D), pa, pd)


def kernel(x, edge_index, edge_type, rel_weight, self_w, self_b):
    src = edge_index[0].astype(jnp.int32)
    dst = edge_index[1].astype(jnp.int32)
    et = edge_type.astype(jnp.int32)

    pad = EPAD - E
    zpad = jnp.zeros((pad,), jnp.int32)
    src2 = jnp.concatenate([src, zpad]).reshape(NS, EPW)
    et2 = jnp.concatenate([et, zpad]).reshape(NS, EPW)
    dst3 = jnp.concatenate([dst, jnp.full((pad,), NP - 1, jnp.int32)]
                           ).reshape(NS, NCHUNK, C)

    ia2, ib2 = _edge_prep(src2, et2)
    y = _compute_y(x, rel_weight)
    yf = y.reshape(2 * R * N, DH)

    z2 = jnp.zeros((NP, DH), jnp.float32)
    z1 = jnp.zeros((NP,), jnp.float32)
    pa, pd = _sc_aggregate(yf, ia2, ib2, dst3, z2, z1)

    return _combine(x, self_w, self_b, pa, pd.reshape(NP, 1))


# speedup vs baseline: 33.1788x; 1.1299x over previous
import jax
import jax.numpy as jnp
from jax import lax
from jax.experimental import pallas as pl
from jax.experimental.pallas import tpu as pltpu
from jax.experimental.pallas import tpu_sc as plsc

N = 10000
E = 320000
D = 128
R = 8

NC = 2
NS = 16
DH = D // NC
C = 80
EPW = E // NS
NCHUNK = EPW // C
NP = 10240
RPT = NP // NS
BN = 1000


def _y_body(x_ref, w2_ref, y_ref):
    y_ref[...] = lax.dot_general(
        x_ref[...], w2_ref[...], (((1,), (0,)), ((), ())),
        preferred_element_type=jnp.float32)


def _compute_y(x, w2):
    return pl.pallas_call(
        _y_body,
        grid=(N // BN,),
        in_specs=[
            pl.BlockSpec((BN, D), lambda i: (i, 0)),
            pl.BlockSpec((D, R * D), lambda i: (0, 0)),
        ],
        out_specs=pl.BlockSpec((BN, R * D), lambda i: (i, 0)),
        out_shape=jax.ShapeDtypeStruct((N, R * D), jnp.float32),
    )(x, w2)


def _edge_prep_body(src_ref, et_ref, ia_ref, ib_ref):
    base = src_ref[...] * (2 * R) + et_ref[...] * 2
    ia_ref[...] = base
    ib_ref[...] = base + 1


def _edge_prep(src2, et2):
    return pl.pallas_call(
        _edge_prep_body,
        out_shape=[
            jax.ShapeDtypeStruct((NS, EPW), jnp.int32),
            jax.ShapeDtypeStruct((NS, EPW), jnp.int32),
        ],
    )(src2, et2)


def _sc_body(y_hbm, ia_hbm, ib_hbm, dst_hbm, z2_hbm, z1_hbm,
             agg_out, deg_out,
             idx_v, dst_v, rows_v, ones_v, agg_sh, deg_sh, sem, sem_s):
    cid = lax.axis_index("c")
    sid = lax.axis_index("s")

    slab = pl.ds(sid * RPT, RPT)
    pltpu.sync_copy(z2_hbm.at[slab], agg_sh.at[slab])

    @pl.when(cid == 1)
    def _():
        pltpu.sync_copy(z1_hbm.at[slab], deg_sh.at[slab])

    @pl.when(cid == 0)
    def _():
        pltpu.sync_copy(ia_hbm.at[sid], idx_v)

    @pl.when(cid == 1)
    def _():
        pltpu.sync_copy(ib_hbm.at[sid], idx_v)

    pltpu.sync_copy(dst_hbm.at[sid], dst_v)

    for i in range(C // 16):
        ones_v[pl.ds(i * 16, 16)] = jnp.ones((16,), jnp.float32)

    plsc.subcore_barrier()

    def run(with_deg):
        def start_gather(j, p):
            pltpu.async_copy(
                y_hbm.at[idx_v.at[pl.ds(j * C, C)]], rows_v.at[p], sem)

        def wait_gather(j, p):
            pltpu.make_async_copy(
                y_hbm.at[idx_v.at[pl.ds(j * C, C)]], rows_v.at[p], sem).wait()

        def start_scatter(j, p):
            pltpu.async_copy(rows_v.at[p], agg_sh.at[dst_v.at[j]], sem_s,
                             add=True)
            if with_deg:
                pltpu.async_copy(ones_v, deg_sh.at[dst_v.at[j]], sem_s,
                                 add=True)

        def wait_scatter(j, p):
            pltpu.make_async_copy(rows_v.at[p], agg_sh.at[dst_v.at[j]],
                                  sem_s).wait()
            if with_deg:
                pltpu.make_async_copy(ones_v, deg_sh.at[dst_v.at[j]],
                                      sem_s).wait()

        start_gather(0, 0)
        start_gather(1, 1)

        def body(j, carry):
            p = lax.bitwise_and(j, 3)
            wait_gather(j, p)
            start_scatter(j, p)
            pl.when(j >= 2)(lambda: wait_scatter(j - 2,
                                                 lax.bitwise_and(j - 2, 3)))
            pl.when(j + 2 < NCHUNK)(
                lambda: start_gather(j + 2, lax.bitwise_and(j + 2, 3)))
            return carry
        lax.fori_loop(0, NCHUNK, body, 0)

        wait_scatter(NCHUNK - 2, (NCHUNK - 2) % 4)
        wait_scatter(NCHUNK - 1, (NCHUNK - 1) % 4)

    @pl.when(cid == 0)
    def _():
        run(False)

    @pl.when(cid == 1)
    def _():
        run(True)

    plsc.subcore_barrier()

    pltpu.sync_copy(agg_sh.at[slab],
                    agg_out.at[slab, pl.ds(cid * DH, DH)])

    @pl.when(cid == 1)
    def _():
        pltpu.sync_copy(deg_sh.at[slab], deg_out.at[slab])


def _sc_aggregate(yf, ia2, ib2, dst3, z2, z1):
    mesh = plsc.VectorSubcoreMesh(core_axis_name="c", subcore_axis_name="s",
                                  num_cores=NC, num_subcores=NS)
    k = pl.kernel(
        _sc_body,
        out_type=(jax.ShapeDtypeStruct((NP, D), jnp.float32),
                  jax.ShapeDtypeStruct((NP,), jnp.float32)),
        mesh=mesh,
        scratch_types=[
            pltpu.VMEM((EPW,), jnp.int32),
            pltpu.VMEM((NCHUNK, C), jnp.int32),
            pltpu.VMEM((4, C, DH), jnp.float32),
            pltpu.VMEM((C,), jnp.float32),
            pltpu.VMEM_SHARED((NP, DH), jnp.float32),
            pltpu.VMEM_SHARED((NP,), jnp.float32),
            pltpu.SemaphoreType.DMA,
            pltpu.SemaphoreType.DMA,
        ],
        compiler_params=pltpu.CompilerParams(use_tc_tiling_on_sc=False),
    )
    return k(yf, ia2, ib2, dst3, z2, z1)


def _combine_body(x_ref, w_ref, b_ref, pa_ref, pd_ref, o_ref):
    deg = jnp.maximum(pd_ref[...], 1.0)
    h = lax.dot_general(x_ref[...], w_ref[...], (((1,), (1,)), ((), ())),
                        preferred_element_type=jnp.float32)
    o_ref[...] = jnp.maximum(h + b_ref[...] + pa_ref[...] / deg, 0.0)


def _combine(x, self_w, self_b, pa, pd):
    return pl.pallas_call(
        _combine_body,
        grid=(N // BN,),
        in_specs=[
            pl.BlockSpec((BN, D), lambda i: (i, 0)),
            pl.BlockSpec((D, D), lambda i: (0, 0)),
            pl.BlockSpec((1, D), lambda i: (0, 0)),
            pl.BlockSpec((BN, D), lambda i: (i, 0)),
            pl.BlockSpec((BN, 1), lambda i: (i, 0)),
        ],
        out_specs=pl.BlockSpec((BN, D), lambda i: (i, 0)),
        out_shape=jax.ShapeDtypeStruct((N, D), jnp.float32),
    )(x, self_w, self_b.reshape(1, D), pa, pd)


def kernel(x, edge_index, edge_type, rel_weight, self_w, self_b):
    src2 = edge_index[0].astype(jnp.int32).reshape(NS, EPW)
    dst3 = edge_index[1].astype(jnp.int32).reshape(NS, NCHUNK, C)
    et2 = edge_type.astype(jnp.int32).reshape(NS, EPW)

    ia2, ib2 = _edge_prep(src2, et2)

    w2 = jnp.transpose(rel_weight, (1, 0, 2)).reshape(D, R * D)
    y = _compute_y(x, w2)
    yf = y.reshape(2 * R * N, DH)

    z2 = jnp.zeros((NP, DH), jnp.float32)
    z1 = jnp.zeros((NP,), jnp.float32)
    pa, pd = _sc_aggregate(yf, ia2, ib2, dst3, z2, z1)

    return _combine(x, self_w, self_b, pa, pd.reshape(NP, 1))


# speedup vs baseline: 33.2517x; 1.0022x over previous
import jax
import jax.numpy as jnp
from jax import lax
from jax.experimental import pallas as pl
from jax.experimental.pallas import tpu as pltpu
from jax.experimental.pallas import tpu_sc as plsc

N = 10000
E = 320000
D = 128
R = 8

NC = 2
NS = 16
DH = D // NC
C = 80
EPW = E // NS
NCHUNK = EPW // C
NP = 10240
RPT = NP // NS
BN = 1000


def _y_body(x_ref, w_ref, y_ref):
    for r in range(R):
        y_ref[:, r, :] = lax.dot_general(
            x_ref[...], w_ref[r], (((1,), (0,)), ((), ())),
            preferred_element_type=jnp.float32)


def _compute_y(x, w):
    return pl.pallas_call(
        _y_body,
        grid=(N // BN,),
        in_specs=[
            pl.BlockSpec((BN, D), lambda i: (i, 0)),
            pl.BlockSpec((R, D, D), lambda i: (0, 0, 0)),
        ],
        out_specs=pl.BlockSpec((BN, R, D), lambda i: (i, 0, 0)),
        out_shape=jax.ShapeDtypeStruct((N, R, D), jnp.float32),
    )(x, w)


def _edge_prep_body(src_ref, et_ref, ia_ref, ib_ref):
    base = src_ref[...] * (2 * R) + et_ref[...] * 2
    ia_ref[...] = base
    ib_ref[...] = base + 1


def _edge_prep(src2, et2):
    return pl.pallas_call(
        _edge_prep_body,
        out_shape=[
            jax.ShapeDtypeStruct((NS, EPW), jnp.int32),
            jax.ShapeDtypeStruct((NS, EPW), jnp.int32),
        ],
    )(src2, et2)


def _sc_body(y_hbm, ia_hbm, ib_hbm, dst_hbm, z2_hbm, z1_hbm,
             agg_out, deg_out,
             idx_v, dst_v, rows_v, ones_v, agg_sh, deg_sh, sem, sem_s):
    cid = lax.axis_index("c")
    sid = lax.axis_index("s")

    slab = pl.ds(sid * RPT, RPT)
    pltpu.sync_copy(z2_hbm.at[slab], agg_sh.at[slab])

    @pl.when(cid == 1)
    def _():
        pltpu.sync_copy(z1_hbm.at[slab], deg_sh.at[slab])

    @pl.when(cid == 0)
    def _():
        pltpu.sync_copy(ia_hbm.at[sid], idx_v)

    @pl.when(cid == 1)
    def _():
        pltpu.sync_copy(ib_hbm.at[sid], idx_v)

    pltpu.sync_copy(dst_hbm.at[sid], dst_v)

    for i in range(C // 16):
        ones_v[pl.ds(i * 16, 16)] = jnp.ones((16,), jnp.float32)

    plsc.subcore_barrier()

    def run(with_deg):
        def start_gather(j, p):
            pltpu.async_copy(
                y_hbm.at[idx_v.at[pl.ds(j * C, C)]], rows_v.at[p], sem)

        def wait_gather(j, p):
            pltpu.make_async_copy(
                y_hbm.at[idx_v.at[pl.ds(j * C, C)]], rows_v.at[p], sem).wait()

        def start_scatter(j, p):
            pltpu.async_copy(rows_v.at[p], agg_sh.at[dst_v.at[j]], sem_s,
                             add=True)
            if with_deg:
                pltpu.async_copy(ones_v, deg_sh.at[dst_v.at[j]], sem_s,
                                 add=True)

        def wait_scatter(j, p):
            pltpu.make_async_copy(rows_v.at[p], agg_sh.at[dst_v.at[j]],
                                  sem_s).wait()
            if with_deg:
                pltpu.make_async_copy(ones_v, deg_sh.at[dst_v.at[j]],
                                      sem_s).wait()

        start_gather(0, 0)
        start_gather(1, 1)

        def body(j, carry):
            p = lax.bitwise_and(j, 3)
            wait_gather(j, p)
            start_scatter(j, p)
            pl.when(j >= 2)(lambda: wait_scatter(j - 2,
                                                 lax.bitwise_and(j - 2, 3)))
            pl.when(j + 2 < NCHUNK)(
                lambda: start_gather(j + 2, lax.bitwise_and(j + 2, 3)))
            return carry
        lax.fori_loop(0, NCHUNK, body, 0)

        wait_scatter(NCHUNK - 2, (NCHUNK - 2) % 4)
        wait_scatter(NCHUNK - 1, (NCHUNK - 1) % 4)

    @pl.when(cid == 0)
    def _():
        run(False)

    @pl.when(cid == 1)
    def _():
        run(True)

    plsc.subcore_barrier()

    pltpu.sync_copy(agg_sh.at[slab],
                    agg_out.at[slab, pl.ds(cid * DH, DH)])

    @pl.when(cid == 1)
    def _():
        pltpu.sync_copy(deg_sh.at[slab], deg_out.at[slab])


def _sc_aggregate(yf, ia2, ib2, dst3, z2, z1):
    mesh = plsc.VectorSubcoreMesh(core_axis_name="c", subcore_axis_name="s",
                                  num_cores=NC, num_subcores=NS)
    k = pl.kernel(
        _sc_body,
        out_type=(jax.ShapeDtypeStruct((NP, D), jnp.float32),
                  jax.ShapeDtypeStruct((NP,), jnp.float32)),
        mesh=mesh,
        scratch_types=[
            pltpu.VMEM((EPW,), jnp.int32),
            pltpu.VMEM((NCHUNK, C), jnp.int32),
            pltpu.VMEM((4, C, DH), jnp.float32),
            pltpu.VMEM((C,), jnp.float32),
            pltpu.VMEM_SHARED((NP, DH), jnp.float32),
            pltpu.VMEM_SHARED((NP,), jnp.float32),
            pltpu.SemaphoreType.DMA,
            pltpu.SemaphoreType.DMA,
        ],
        compiler_params=pltpu.CompilerParams(use_tc_tiling_on_sc=False),
    )
    return k(yf, ia2, ib2, dst3, z2, z1)


def _combine_body(x_ref, w_ref, b_ref, pa_ref, pd_ref, o_ref):
    deg = jnp.maximum(pd_ref[...], 1.0)
    h = lax.dot_general(x_ref[...], w_ref[...], (((1,), (1,)), ((), ())),
                        preferred_element_type=jnp.float32)
    o_ref[...] = jnp.maximum(h + b_ref[...] + pa_ref[...] / deg, 0.0)


def _combine(x, self_w, self_b, pa, pd):
    return pl.pallas_call(
        _combine_body,
        grid=(N // BN,),
        in_specs=[
            pl.BlockSpec((BN, D), lambda i: (i, 0)),
            pl.BlockSpec((D, D), lambda i: (0, 0)),
            pl.BlockSpec((1, D), lambda i: (0, 0)),
            pl.BlockSpec((BN, D), lambda i: (i, 0)),
            pl.BlockSpec((BN, 1), lambda i: (i, 0)),
        ],
        out_specs=pl.BlockSpec((BN, D), lambda i: (i, 0)),
        out_shape=jax.ShapeDtypeStruct((N, D), jnp.float32),
    )(x, self_w, self_b.reshape(1, D), pa, pd)


def kernel(x, edge_index, edge_type, rel_weight, self_w, self_b):
    src2 = edge_index[0].astype(jnp.int32).reshape(NS, EPW)
    dst3 = edge_index[1].astype(jnp.int32).reshape(NS, NCHUNK, C)
    et2 = edge_type.astype(jnp.int32).reshape(NS, EPW)

    ia2, ib2 = _edge_prep(src2, et2)

    y = _compute_y(x, rel_weight)
    yf = y.reshape(2 * R * N, DH)

    z2 = jnp.zeros((NP, DH), jnp.float32)
    z1 = jnp.zeros((NP,), jnp.float32)
    pa, pd = _sc_aggregate(yf, ia2, ib2, dst3, z2, z1)

    return _combine(x, self_w, self_b, pa, pd.reshape(NP, 1))


# speedup vs baseline: 34.8225x; 1.0472x over previous
import jax
import jax.numpy as jnp
from jax import lax
from jax.experimental import pallas as pl
from jax.experimental.pallas import tpu as pltpu
from jax.experimental.pallas import tpu_sc as plsc

N = 10000
E = 320000
D = 128
R = 8

NC = 2
NS = 16
DH = D // NC
C = 80
EPW = E // NS
NCHUNK = EPW // C
NP = 10240
RPT = NP // NS
BN = 1000


def _y_body(x_ref, w_ref, y_ref):
    for r in range(R):
        y_ref[r] = lax.dot_general(
            x_ref[...], w_ref[r], (((1,), (0,)), ((), ())),
            preferred_element_type=jnp.float32)


def _compute_y(x, w):
    return pl.pallas_call(
        _y_body,
        grid=(N // BN,),
        in_specs=[
            pl.BlockSpec((BN, D), lambda i: (i, 0)),
            pl.BlockSpec((R, D, D), lambda i: (0, 0, 0)),
        ],
        out_specs=pl.BlockSpec((R, BN, D), lambda i: (0, i, 0)),
        out_shape=jax.ShapeDtypeStruct((R, N, D), jnp.float32),
    )(x, w)


def _edge_prep_body(src_ref, et_ref, ia_ref, ib_ref):
    base = (et_ref[...] * N + src_ref[...]) * 2
    ia_ref[...] = base
    ib_ref[...] = base + 1


def _edge_prep(src2, et2):
    return pl.pallas_call(
        _edge_prep_body,
        out_shape=[
            jax.ShapeDtypeStruct((NS, EPW), jnp.int32),
            jax.ShapeDtypeStruct((NS, EPW), jnp.int32),
        ],
    )(src2, et2)


def _sc_body(y_hbm, ia_hbm, ib_hbm, dst_hbm, z2_hbm, z1_hbm,
             agg_out, deg_out,
             idx_v, dst_v, rows_v, ones_v, agg_sh, deg_sh, sem, sem_s):
    cid = lax.axis_index("c")
    sid = lax.axis_index("s")

    slab = pl.ds(sid * RPT, RPT)
    pltpu.sync_copy(z2_hbm.at[slab], agg_sh.at[slab])

    @pl.when(cid == 1)
    def _():
        pltpu.sync_copy(z1_hbm.at[slab], deg_sh.at[slab])

    @pl.when(cid == 0)
    def _():
        pltpu.sync_copy(ia_hbm.at[sid], idx_v)

    @pl.when(cid == 1)
    def _():
        pltpu.sync_copy(ib_hbm.at[sid], idx_v)

    pltpu.sync_copy(dst_hbm.at[sid], dst_v)

    for i in range(C // 16):
        ones_v[pl.ds(i * 16, 16)] = jnp.ones((16,), jnp.float32)

    plsc.subcore_barrier()

    def run(with_deg):
        def start_gather(j, p):
            pltpu.async_copy(
                y_hbm.at[idx_v.at[pl.ds(j * C, C)]], rows_v.at[p], sem)

        def wait_gather(j, p):
            pltpu.make_async_copy(
                y_hbm.at[idx_v.at[pl.ds(j * C, C)]], rows_v.at[p], sem).wait()

        def start_scatter(j, p):
            pltpu.async_copy(rows_v.at[p], agg_sh.at[dst_v.at[j]], sem_s,
                             add=True)
            if with_deg:
                pltpu.async_copy(ones_v, deg_sh.at[dst_v.at[j]], sem_s,
                                 add=True)

        def wait_scatter(j, p):
            pltpu.make_async_copy(rows_v.at[p], agg_sh.at[dst_v.at[j]],
                                  sem_s).wait()
            if with_deg:
                pltpu.make_async_copy(ones_v, deg_sh.at[dst_v.at[j]],
                                      sem_s).wait()

        start_gather(0, 0)
        start_gather(1, 1)

        def body(j, carry):
            p = lax.bitwise_and(j, 3)
            wait_gather(j, p)
            start_scatter(j, p)
            pl.when(j >= 2)(lambda: wait_scatter(j - 2,
                                                 lax.bitwise_and(j - 2, 3)))
            pl.when(j + 2 < NCHUNK)(
                lambda: start_gather(j + 2, lax.bitwise_and(j + 2, 3)))
            return carry
        lax.fori_loop(0, NCHUNK, body, 0)

        wait_scatter(NCHUNK - 2, (NCHUNK - 2) % 4)
        wait_scatter(NCHUNK - 1, (NCHUNK - 1) % 4)

    @pl.when(cid == 0)
    def _():
        run(False)

    @pl.when(cid == 1)
    def _():
        run(True)

    plsc.subcore_barrier()

    pltpu.sync_copy(agg_sh.at[slab],
                    agg_out.at[slab, pl.ds(cid * DH, DH)])

    @pl.when(cid == 1)
    def _():
        pltpu.sync_copy(deg_sh.at[slab], deg_out.at[slab])


def _sc_aggregate(yf, ia2, ib2, dst3, z2, z1):
    mesh = plsc.VectorSubcoreMesh(core_axis_name="c", subcore_axis_name="s",
                                  num_cores=NC, num_subcores=NS)
    k = pl.kernel(
        _sc_body,
        out_type=(jax.ShapeDtypeStruct((NP, D), jnp.float32),
                  jax.ShapeDtypeStruct((NP,), jnp.float32)),
        mesh=mesh,
        scratch_types=[
            pltpu.VMEM((EPW,), jnp.int32),
            pltpu.VMEM((NCHUNK, C), jnp.int32),
            pltpu.VMEM((4, C, DH), jnp.float32),
            pltpu.VMEM((C,), jnp.float32),
            pltpu.VMEM_SHARED((NP, DH), jnp.float32),
            pltpu.VMEM_SHARED((NP,), jnp.float32),
            pltpu.SemaphoreType.DMA,
            pltpu.SemaphoreType.DMA,
        ],
        compiler_params=pltpu.CompilerParams(use_tc_tiling_on_sc=False),
    )
    return k(yf, ia2, ib2, dst3, z2, z1)


def _combine_body(x_ref, w_ref, b_ref, pa_ref, pd_ref, o_ref):
    deg = jnp.maximum(pd_ref[...], 1.0)
    h = lax.dot_general(x_ref[...], w_ref[...], (((1,), (1,)), ((), ())),
                        preferred_element_type=jnp.float32)
    o_ref[...] = jnp.maximum(h + b_ref[...] + pa_ref[...] / deg, 0.0)


def _combine(x, self_w, self_b, pa, pd):
    return pl.pallas_call(
        _combine_body,
        grid=(N // BN,),
        in_specs=[
            pl.BlockSpec((BN, D), lambda i: (i, 0)),
            pl.BlockSpec((D, D), lambda i: (0, 0)),
            pl.BlockSpec((1, D), lambda i: (0, 0)),
            pl.BlockSpec((BN, D), lambda i: (i, 0)),
            pl.BlockSpec((BN, 1), lambda i: (i, 0)),
        ],
        out_specs=pl.BlockSpec((BN, D), lambda i: (i, 0)),
        out_shape=jax.ShapeDtypeStruct((N, D), jnp.float32),
    )(x, self_w, self_b.reshape(1, D), pa, pd)


def kernel(x, edge_index, edge_type, rel_weight, self_w, self_b):
    src2 = edge_index[0].astype(jnp.int32).reshape(NS, EPW)
    dst3 = edge_index[1].astype(jnp.int32).reshape(NS, NCHUNK, C)
    et2 = edge_type.astype(jnp.int32).reshape(NS, EPW)

    ia2, ib2 = _edge_prep(src2, et2)

    y = _compute_y(x, rel_weight)
    yf = y.reshape(2 * R * N, DH)

    z2 = jnp.zeros((NP, DH), jnp.float32)
    z1 = jnp.zeros((NP,), jnp.float32)
    pa, pd = _sc_aggregate(yf, ia2, ib2, dst3, z2, z1)

    return _combine(x, self_w, self_b, pa, pd.reshape(NP, 1))


# speedup vs baseline: 39.0066x; 1.1202x over previous
import jax
import jax.numpy as jnp
from jax import lax
from jax.experimental import pallas as pl
from jax.experimental.pallas import tpu as pltpu
from jax.experimental.pallas import tpu_sc as plsc

N = 10000
E = 320000
D = 128
R = 8

NC = 2
NS = 16
DH = D // NC
C = 80
EPW = E // NS
NCHUNK = EPW // C
NP = 10240
RPT = NP // NS
BN = 1000


def _y_body(x_ref, w_ref, y_ref):
    for r in range(R):
        y_ref[r] = lax.dot_general(
            x_ref[...], w_ref[r], (((1,), (0,)), ((), ())),
            preferred_element_type=jnp.float32)


def _compute_y(x, w):
    return pl.pallas_call(
        _y_body,
        grid=(N // BN,),
        in_specs=[
            pl.BlockSpec((BN, D), lambda i: (i, 0)),
            pl.BlockSpec((R, D, D), lambda i: (0, 0, 0)),
        ],
        out_specs=pl.BlockSpec((R, BN, D), lambda i: (0, i, 0)),
        out_shape=jax.ShapeDtypeStruct((R, N, D), jnp.float32),
    )(x, w)


EB = E // 10


def _edge_prep_body(ei_ref, et_ref, ia_ref, ib_ref, dst_ref):
    base = (et_ref[...] * N + ei_ref[0]) * 2
    ia_ref[...] = base
    ib_ref[...] = base + 1
    dst_ref[...] = ei_ref[1]


def _edge_prep(ei, et):
    return pl.pallas_call(
        _edge_prep_body,
        out_shape=[
            jax.ShapeDtypeStruct((E,), jnp.int32),
            jax.ShapeDtypeStruct((E,), jnp.int32),
            jax.ShapeDtypeStruct((E,), jnp.int32),
        ],
    )(ei, et)


def _sc_body(y_hbm, ia_hbm, ib_hbm, dst_hbm, z2_hbm, z1_hbm,
             agg_out, deg_out,
             idx_v, dst_v, rows_v, ones_v, agg_sh, deg_sh, sem, sem_s):
    cid = lax.axis_index("c")
    sid = lax.axis_index("s")

    slab = pl.ds(sid * RPT, RPT)
    pltpu.sync_copy(z2_hbm.at[slab], agg_sh.at[slab])

    @pl.when(cid == 1)
    def _():
        pltpu.sync_copy(z1_hbm.at[slab], deg_sh.at[slab])

    eslab = pl.ds(sid * EPW, EPW)

    @pl.when(cid == 0)
    def _():
        pltpu.sync_copy(ia_hbm.at[eslab], idx_v)

    @pl.when(cid == 1)
    def _():
        pltpu.sync_copy(ib_hbm.at[eslab], idx_v)

    pltpu.sync_copy(dst_hbm.at[eslab], dst_v)

    def dlist(j):
        return dst_v.at[pl.ds(j * C, C)]

    for i in range(C // 16):
        ones_v[pl.ds(i * 16, 16)] = jnp.ones((16,), jnp.float32)

    plsc.subcore_barrier()

    def run(with_deg):
        def start_gather(j, p):
            pltpu.async_copy(
                y_hbm.at[idx_v.at[pl.ds(j * C, C)]], rows_v.at[p], sem)

        def wait_gather(j, p):
            pltpu.make_async_copy(
                y_hbm.at[idx_v.at[pl.ds(j * C, C)]], rows_v.at[p], sem).wait()

        def start_scatter(j, p):
            pltpu.async_copy(rows_v.at[p], agg_sh.at[dlist(j)], sem_s,
                             add=True)
            if with_deg:
                pltpu.async_copy(ones_v, deg_sh.at[dlist(j)], sem_s,
                                 add=True)

        def wait_scatter(j, p):
            pltpu.make_async_copy(rows_v.at[p], agg_sh.at[dlist(j)],
                                  sem_s).wait()
            if with_deg:
                pltpu.make_async_copy(ones_v, deg_sh.at[dlist(j)],
                                      sem_s).wait()

        start_gather(0, 0)
        start_gather(1, 1)

        def body(j, carry):
            p = lax.bitwise_and(j, 3)
            wait_gather(j, p)
            start_scatter(j, p)
            pl.when(j >= 2)(lambda: wait_scatter(j - 2,
                                                 lax.bitwise_and(j - 2, 3)))
            pl.when(j + 2 < NCHUNK)(
                lambda: start_gather(j + 2, lax.bitwise_and(j + 2, 3)))
            return carry
        lax.fori_loop(0, NCHUNK, body, 0)

        wait_scatter(NCHUNK - 2, (NCHUNK - 2) % 4)
        wait_scatter(NCHUNK - 1, (NCHUNK - 1) % 4)

    @pl.when(cid == 0)
    def _():
        run(False)

    @pl.when(cid == 1)
    def _():
        run(True)

    plsc.subcore_barrier()

    pltpu.sync_copy(agg_sh.at[slab],
                    agg_out.at[slab, pl.ds(cid * DH, DH)])

    @pl.when(cid == 1)
    def _():
        pltpu.sync_copy(deg_sh.at[slab], deg_out.at[slab])


def _sc_aggregate(yf, ia2, ib2, dst2, z2, z1):
    mesh = plsc.VectorSubcoreMesh(core_axis_name="c", subcore_axis_name="s",
                                  num_cores=NC, num_subcores=NS)
    k = pl.kernel(
        _sc_body,
        out_type=(jax.ShapeDtypeStruct((NP, D), jnp.float32),
                  jax.ShapeDtypeStruct((NP,), jnp.float32)),
        mesh=mesh,
        scratch_types=[
            pltpu.VMEM((EPW,), jnp.int32),
            pltpu.VMEM((EPW,), jnp.int32),
            pltpu.VMEM((4, C, DH), jnp.float32),
            pltpu.VMEM((C,), jnp.float32),
            pltpu.VMEM_SHARED((NP, DH), jnp.float32),
            pltpu.VMEM_SHARED((NP,), jnp.float32),
            pltpu.SemaphoreType.DMA,
            pltpu.SemaphoreType.DMA,
        ],
        compiler_params=pltpu.CompilerParams(use_tc_tiling_on_sc=False),
    )
    return k(yf, ia2, ib2, dst2, z2, z1)


def _combine_body(x_ref, w_ref, b_ref, pa_ref, pd_ref, o_ref):
    deg = jnp.maximum(pd_ref[...], 1.0)
    h = lax.dot_general(x_ref[...], w_ref[...], (((1,), (1,)), ((), ())),
                        preferred_element_type=jnp.float32)
    o_ref[...] = jnp.maximum(h + b_ref[...] + pa_ref[...] / deg, 0.0)


def _combine(x, self_w, self_b, pa, pd):
    return pl.pallas_call(
        _combine_body,
        grid=(N // BN,),
        in_specs=[
            pl.BlockSpec((BN, D), lambda i: (i, 0)),
            pl.BlockSpec((D, D), lambda i: (0, 0)),
            pl.BlockSpec((1, D), lambda i: (0, 0)),
            pl.BlockSpec((BN, D), lambda i: (i, 0)),
            pl.BlockSpec((BN, 1), lambda i: (i, 0)),
        ],
        out_specs=pl.BlockSpec((BN, D), lambda i: (i, 0)),
        out_shape=jax.ShapeDtypeStruct((N, D), jnp.float32),
    )(x, self_w, self_b.reshape(1, D), pa, pd)


def kernel(x, edge_index, edge_type, rel_weight, self_w, self_b):
    ei = edge_index.astype(jnp.int32)
    et1 = edge_type.astype(jnp.int32)

    ia2, ib2, dst2 = _edge_prep(ei, et1)

    y = _compute_y(x, rel_weight)
    yf = y.reshape(2 * R * N, DH)

    z2 = jnp.zeros((NP, DH), jnp.float32)
    z1 = jnp.zeros((NP,), jnp.float32)
    pa, pd = _sc_aggregate(yf, ia2, ib2, dst2, z2, z1)

    return _combine(x, self_w, self_b, pa, pd.reshape(NP, 1))


# speedup vs baseline: 44.5852x; 1.1430x over previous
import jax
import jax.numpy as jnp
from jax import lax
from jax.experimental import pallas as pl
from jax.experimental.pallas import tpu as pltpu
from jax.experimental.pallas import tpu_sc as plsc

N = 10000
E = 320000
D = 128
R = 8

NC = 2
NS = 16
DH = D // NC
C = 128
EPW = E // NS
NCHUNK = EPW // C
TAIL = EPW - NCHUNK * C
NP = 10240
RPT = NP // NS
BN = 1000


def _y_body(x_ref, w2_ref, y_ref):
    h = lax.dot_general(
        x_ref[...], w2_ref[...], (((1,), (0,)), ((), ())),
        preferred_element_type=jnp.float32)
    for r in range(R):
        y_ref[r] = h[:, r * D:(r + 1) * D]


def _compute_y(x, w2):
    return pl.pallas_call(
        _y_body,
        grid=(N // BN,),
        in_specs=[
            pl.BlockSpec((BN, D), lambda i: (i, 0)),
            pl.BlockSpec((D, R * D), lambda i: (0, 0)),
        ],
        out_specs=pl.BlockSpec((R, BN, D), lambda i: (0, i, 0)),
        out_shape=jax.ShapeDtypeStruct((R, N, D), jnp.float32),
    )(x, w2)


EB = E // 10


def _edge_prep_body(ei_ref, et_ref, ia_ref, ib_ref, dst_ref):
    base = (et_ref[...] * N + ei_ref[0]) * 2
    ia_ref[...] = base
    ib_ref[...] = base + 1
    dst_ref[...] = ei_ref[1]


def _edge_prep(ei, et):
    return pl.pallas_call(
        _edge_prep_body,
        out_shape=[
            jax.ShapeDtypeStruct((E,), jnp.int32),
            jax.ShapeDtypeStruct((E,), jnp.int32),
            jax.ShapeDtypeStruct((E,), jnp.int32),
        ],
    )(ei, et)


def _sc_body(y_hbm, ia_hbm, ib_hbm, dst_hbm, z2_hbm, z1_hbm,
             agg_out, deg_out,
             idx_v, dst_v, rows_v, ones_v, agg_sh, deg_sh, sem, sem_s):
    cid = lax.axis_index("c")
    sid = lax.axis_index("s")

    slab = pl.ds(sid * RPT, RPT)
    pltpu.sync_copy(z2_hbm.at[slab], agg_sh.at[slab])

    @pl.when(cid == 1)
    def _():
        pltpu.sync_copy(z1_hbm.at[slab], deg_sh.at[slab])

    eslab = pl.ds(sid * EPW, EPW)

    @pl.when(cid == 0)
    def _():
        pltpu.sync_copy(ia_hbm.at[eslab], idx_v)

    @pl.when(cid == 1)
    def _():
        pltpu.sync_copy(ib_hbm.at[eslab], idx_v)

    pltpu.sync_copy(dst_hbm.at[eslab], dst_v)

    def dlist(j):
        return dst_v.at[pl.ds(j * C, C)]

    for i in range(C // 16):
        ones_v[pl.ds(i * 16, 16)] = jnp.ones((16,), jnp.float32)

    plsc.subcore_barrier()

    def run(with_deg):
        def start_gather(j, p):
            pltpu.async_copy(
                y_hbm.at[idx_v.at[pl.ds(j * C, C)]], rows_v.at[p], sem)

        def wait_gather(j, p):
            pltpu.make_async_copy(
                y_hbm.at[idx_v.at[pl.ds(j * C, C)]], rows_v.at[p], sem).wait()

        def start_scatter(j, p):
            pltpu.async_copy(rows_v.at[p], agg_sh.at[dlist(j)], sem_s,
                             add=True)
            if with_deg:
                pltpu.async_copy(ones_v, deg_sh.at[dlist(j)], sem_s,
                                 add=True)

        def wait_scatter(j, p):
            pltpu.make_async_copy(rows_v.at[p], agg_sh.at[dlist(j)],
                                  sem_s).wait()
            if with_deg:
                pltpu.make_async_copy(ones_v, deg_sh.at[dlist(j)],
                                      sem_s).wait()

        start_gather(0, 0)
        start_gather(1, 1)

        def body(j, carry):
            p = lax.bitwise_and(j, 3)
            wait_gather(j, p)
            start_scatter(j, p)
            pl.when(j >= 2)(lambda: wait_scatter(j - 2,
                                                 lax.bitwise_and(j - 2, 3)))
            pl.when(j + 2 < NCHUNK)(
                lambda: start_gather(j + 2, lax.bitwise_and(j + 2, 3)))
            return carry
        lax.fori_loop(0, NCHUNK, body, 0)

        wait_scatter(NCHUNK - 2, (NCHUNK - 2) % 4)
        wait_scatter(NCHUNK - 1, (NCHUNK - 1) % 4)

        tslab = pl.ds(NCHUNK * C, TAIL)
        pltpu.async_copy(y_hbm.at[idx_v.at[tslab]],
                         rows_v.at[0, pl.ds(0, TAIL)], sem)
        pltpu.make_async_copy(y_hbm.at[idx_v.at[tslab]],
                              rows_v.at[0, pl.ds(0, TAIL)], sem).wait()
        pltpu.sync_copy(rows_v.at[0, pl.ds(0, TAIL)],
                        agg_sh.at[dst_v.at[tslab]], add=True)
        if with_deg:
            pltpu.sync_copy(ones_v.at[pl.ds(0, TAIL)],
                            deg_sh.at[dst_v.at[tslab]], add=True)

    @pl.when(cid == 0)
    def _():
        run(False)

    @pl.when(cid == 1)
    def _():
        run(True)

    plsc.subcore_barrier()

    pltpu.sync_copy(agg_sh.at[slab],
                    agg_out.at[slab, pl.ds(cid * DH, DH)])

    @pl.when(cid == 1)
    def _():
        pltpu.sync_copy(deg_sh.at[slab], deg_out.at[slab])


def _sc_aggregate(yf, ia2, ib2, dst2, z2, z1):
    mesh = plsc.VectorSubcoreMesh(core_axis_name="c", subcore_axis_name="s",
                                  num_cores=NC, num_subcores=NS)
    k = pl.kernel(
        _sc_body,
        out_type=(jax.ShapeDtypeStruct((NP, D), jnp.float32),
                  jax.ShapeDtypeStruct((NP,), jnp.float32)),
        mesh=mesh,
        scratch_types=[
            pltpu.VMEM((EPW,), jnp.int32),
            pltpu.VMEM((EPW,), jnp.int32),
            pltpu.VMEM((4, C, DH), jnp.float32),
            pltpu.VMEM((C,), jnp.float32),
            pltpu.VMEM_SHARED((NP, DH), jnp.float32),
            pltpu.VMEM_SHARED((NP,), jnp.float32),
            pltpu.SemaphoreType.DMA,
            pltpu.SemaphoreType.DMA,
        ],
        compiler_params=pltpu.CompilerParams(use_tc_tiling_on_sc=False),
    )
    return k(yf, ia2, ib2, dst2, z2, z1)


def _combine_body(x_ref, w_ref, b_ref, pa_ref, pd_ref, o_ref):
    deg = jnp.maximum(pd_ref[...], 1.0)
    h = lax.dot_general(x_ref[...], w_ref[...], (((1,), (1,)), ((), ())),
                        preferred_element_type=jnp.float32)
    o_ref[...] = jnp.maximum(h + b_ref[...] + pa_ref[...] / deg, 0.0)


def _combine(x, self_w, self_b, pa, pd):
    return pl.pallas_call(
        _combine_body,
        grid=(N // BN,),
        in_specs=[
            pl.BlockSpec((BN, D), lambda i: (i, 0)),
            pl.BlockSpec((D, D), lambda i: (0, 0)),
            pl.BlockSpec((1, D), lambda i: (0, 0)),
            pl.BlockSpec((BN, D), lambda i: (i, 0)),
            pl.BlockSpec((BN, 1), lambda i: (i, 0)),
        ],
        out_specs=pl.BlockSpec((BN, D), lambda i: (i, 0)),
        out_shape=jax.ShapeDtypeStruct((N, D), jnp.float32),
    )(x, self_w, self_b.reshape(1, D), pa, pd)


def kernel(x, edge_index, edge_type, rel_weight, self_w, self_b):
    ei = edge_index.astype(jnp.int32)
    et1 = edge_type.astype(jnp.int32)

    ia2, ib2, dst2 = _edge_prep(ei, et1)

    w2 = jnp.transpose(rel_weight, (1, 0, 2)).reshape(D, R * D)
    y = _compute_y(x, w2)
    yf = y.reshape(2 * R * N, DH)

    z2 = jnp.zeros((NP, DH), jnp.float32)
    z1 = jnp.zeros((NP,), jnp.float32)
    pa, pd = _sc_aggregate(yf, ia2, ib2, dst2, z2, z1)

    return _combine(x, self_w, self_b, pa, pd.reshape(NP, 1))
